# Initial kernel scaffold; baseline (speedup 1.0000x reference)
import jax
import jax.numpy as jnp
from jax.experimental import pallas as pl


def kernel(token_ids, embd_mat):
    raise NotImplementedError("write your pallas kernel here")


# speedup vs baseline: 1.7972x; 1.7972x over previous
import functools

import jax
import jax.numpy as jnp
from jax import lax
from jax.experimental import pallas as pl
from jax.experimental.pallas import tpu as pltpu
from jax.experimental.pallas import tpu_sc as plsc

D = 64
B = 16384 * 50
NW = 32
BPW = B // NW
CHUNK = 512
NCHUNK = BPW // CHUNK
KSUB = CHUNK // 128

_mesh = plsc.VectorSubcoreMesh(core_axis_name="c", subcore_axis_name="s")


@functools.partial(
    pl.kernel,
    out_type=jax.ShapeDtypeStruct((B, D), jnp.float32),
    mesh=_mesh,
    scratch_types=[
        pltpu.VMEM((KSUB, 128), jnp.int32),
        pltpu.VMEM((CHUNK, D), jnp.float32),
        pltpu.SemaphoreType.DMA,
    ],
    compiler_params=pltpu.CompilerParams(use_tc_tiling_on_sc=False),
)
def _sc_gather(ids_hbm, table_hbm, out_hbm, idx_v, rows_v, sem):
    wid = lax.axis_index("s") * 2 + lax.axis_index("c")
    row_base = wid * (BPW // 128)

    def step(ci, _):
        pltpu.sync_copy(ids_hbm.at[pl.ds(row_base + ci * KSUB, KSUB)], idx_v)
        copies = [
            pltpu.async_copy(
                table_hbm.at[idx_v.at[j]],
                rows_v.at[pl.ds(j * 128, 128)],
                sem,
            )
            for j in range(KSUB)
        ]
        for cp in copies:
            cp.wait()
        out_off = wid * BPW + ci * CHUNK
        pltpu.sync_copy(rows_v, out_hbm.at[pl.ds(out_off, CHUNK)])
        return 0

    lax.fori_loop(0, NCHUNK, step, 0)


def kernel(token_ids, embd_mat):
    ids = token_ids.reshape(B // 128, 128)
    out = _sc_gather(ids, embd_mat)
    return out.reshape(token_ids.shape[0], token_ids.shape[1], D)


# speedup vs baseline: 1.8739x; 1.0427x over previous
import functools

import jax
import jax.numpy as jnp
from jax import lax
from jax.experimental import pallas as pl
from jax.experimental.pallas import tpu as pltpu
from jax.experimental.pallas import tpu_sc as plsc

D = 64
B = 16384 * 50
NW = 32
BPW = B // NW
CHUNK = 512
NCHUNK = BPW // CHUNK
KSUB = CHUNK // 128
NPAIR = NCHUNK // 2

_mesh = plsc.VectorSubcoreMesh(core_axis_name="c", subcore_axis_name="s")


@functools.partial(
    pl.kernel,
    out_type=jax.ShapeDtypeStruct((B, D), jnp.float32),
    mesh=_mesh,
    scratch_types=[
        pltpu.VMEM((KSUB, 128), jnp.int32),
        pltpu.VMEM((KSUB, 128), jnp.int32),
        pltpu.VMEM((CHUNK, D), jnp.float32),
        pltpu.VMEM((CHUNK, D), jnp.float32),
        pltpu.SemaphoreType.DMA,
        pltpu.SemaphoreType.DMA,
        pltpu.SemaphoreType.DMA,
        pltpu.SemaphoreType.DMA,
    ],
    compiler_params=pltpu.CompilerParams(use_tc_tiling_on_sc=False),
)
def _sc_gather(ids_hbm, table_hbm, out_hbm, idx_a, idx_b, rows_a, rows_b,
               sem_ga, sem_gb, sem_wa, sem_wb):
    wid = lax.axis_index("s") * 2 + lax.axis_index("c")
    row_base = wid * (BPW // 128)
    out_base = wid * BPW

    def stage_idx(ci, idx_v):
        pltpu.sync_copy(ids_hbm.at[pl.ds(row_base + ci * KSUB, KSUB)], idx_v)

    def fire_gathers(idx_v, rows_v, sem):
        return [
            pltpu.async_copy(
                table_hbm.at[idx_v.at[j]],
                rows_v.at[pl.ds(j * 128, 128)],
                sem,
            )
            for j in range(KSUB)
        ]

    def writeout(ci, rows_v, sem):
        return pltpu.make_async_copy(
            rows_v, out_hbm.at[pl.ds(out_base + ci * CHUNK, CHUNK)], sem)

    stage_idx(0, idx_a)
    ga = fire_gathers(idx_a, rows_a, sem_ga)
    stage_idx(1, idx_b)
    gb = fire_gathers(idx_b, rows_b, sem_gb)
    for cp in ga:
        cp.wait()
    wa = writeout(0, rows_a, sem_wa)
    wa.start()
    for cp in gb:
        cp.wait()
    wb = writeout(1, rows_b, sem_wb)
    wb.start()

    def step(g, _):
        c0 = 2 * g
        c1 = c0 + 1
        stage_idx(c0, idx_a)
        writeout(c0 - 2, rows_a, sem_wa).wait()
        ga = fire_gathers(idx_a, rows_a, sem_ga)
        stage_idx(c1, idx_b)
        writeout(c1 - 2, rows_b, sem_wb).wait()
        gb = fire_gathers(idx_b, rows_b, sem_gb)
        for cp in ga:
            cp.wait()
        writeout(c0, rows_a, sem_wa).start()
        for cp in gb:
            cp.wait()
        writeout(c1, rows_b, sem_wb).start()
        return 0

    lax.fori_loop(1, NPAIR, step, 0)

    writeout(NCHUNK - 2, rows_a, sem_wa).wait()
    writeout(NCHUNK - 1, rows_b, sem_wb).wait()


def kernel(token_ids, embd_mat):
    ids = token_ids.reshape(B // 128, 128)
    out = _sc_gather(ids, embd_mat)
    return out.reshape(token_ids.shape[0], token_ids.shape[1], D)
